# baseline (speedup 1.0000x reference)
import jax
import jax.numpy as jnp
from jax import lax
from jax.experimental import pallas as pl
from jax.experimental.pallas import tpu as pltpu
from jax.experimental.pallas import tpu_sc as plsc

M = 100000
D = 128
B = 16384
C = 100
L = 16
NW = 32
RW = 3200
CH = 96
NBUF = 3
NTOT = 36
NCHUNK = B // L
LVREG = CH // L
CAPC = CH + L


def _body(bx_h, lg_h, by_h, bt_h, x_h, lnp_h, byn_h, idx_h, tv_h,
          out_bx, out_by, out_bt, out_lg,
          idx_v, byn_v, win_v, loc_v, src_v,
          cpx0, cpx1, cpx2, cpl0, cpl1, cpl2,
          cby0, cby1, cby2, cbt0, cbt1, cbt2,
          gxb, glb, tbuf,
          si0, si1, si2, so0, so1, so2, sem_g):
    cid = lax.axis_index("c")
    sid = lax.axis_index("s")
    wid = cid * 16 + sid
    lo = wid * RW
    hi = jnp.minimum(lo + RW, M)
    iota = lax.iota(jnp.int32, L)

    cpx = (cpx0, cpx1, cpx2)
    cpl = (cpl0, cpl1, cpl2)
    cby = (cby0, cby1, cby2)
    cbt = (cbt0, cbt1, cbt2)
    sin = (si0, si1, si2)
    sout = (so0, so1, so2)

    pltpu.sync_copy(idx_h, idx_v)
    pltpu.sync_copy(byn_h, byn_v)
    pltpu.sync_copy(tv_h, tbuf)
    tvec = tbuf[...]

    neg1 = jnp.full((L,), -1, jnp.int32)

    def init_b(k, carry):
        win_v[pl.ds(k * L, L)] = neg1
        return carry
    lax.fori_loop(0, RW // L, init_b, 0)

    def scan_b(k, carry):
        v = idx_v[pl.ds(k * L, L)]
        m = (v >= lo) & (v < hi)
        cnt = jnp.sum(m.astype(jnp.int32))
        local = jnp.where(m, v - lo, 0)
        gi = k * L + iota

        @pl.when(cnt == 1)
        def _():
            plsc.store_scatter(win_v, [local], gi, mask=m)

        @pl.when(cnt > 1)
        def _():
            for j in range(L):
                plsc.store_scatter(win_v, [local], gi, mask=m & (iota == j))
        return carry
    lax.fori_loop(0, NCHUNK, scan_b, 0)

    def chunk_base(k):
        return pl.multiple_of(jnp.minimum(lo + k * CH, hi - CH), 8)

    def in_copies(k, b):
        r = chunk_base(k)
        return (
            pltpu.make_async_copy(bx_h.at[pl.ds(r, CH)], cpx[b], sin[b]),
            pltpu.make_async_copy(lg_h.at[pl.ds(r, CH)], cpl[b], sin[b]),
            pltpu.make_async_copy(by_h.at[pl.ds(r, CH)],
                                  cby[b].at[pl.ds(0, CH)], sin[b]),
            pltpu.make_async_copy(bt_h.at[pl.ds(r, CH)],
                                  cbt[b].at[pl.ds(0, CH)], sin[b]),
        )

    def out_copies(k, b):
        r = chunk_base(k)
        return (
            pltpu.make_async_copy(cpx[b], out_bx.at[pl.ds(r, CH)], sout[b]),
            pltpu.make_async_copy(cpl[b], out_lg.at[pl.ds(r, CH)], sout[b]),
            pltpu.make_async_copy(cby[b].at[pl.ds(0, CH)],
                                  out_by.at[pl.ds(r, CH)], sout[b]),
            pltpu.make_async_copy(cbt[b].at[pl.ds(0, CH)],
                                  out_bt.at[pl.ds(r, CH)], sout[b]),
        )

    def start_all(cps):
        for cp in cps:
            cp.start()

    def wait_all(cps):
        for cp in cps:
            cp.wait()

    def update(k, b):
        lb = chunk_base(k) - lo

        def comp_b(j, off):
            w = win_v[pl.ds(lb + j * L, L)]
            mv = w >= 0
            plsc.store_compressed(loc_v.at[pl.ds(off, L)], j * L + iota,
                                  mask=mv)
            plsc.store_compressed(src_v.at[pl.ds(off, L)], w, mask=mv)
            return off + jnp.sum(mv.astype(jnp.int32))
        n_upd = lax.fori_loop(0, LVREG, comp_b, jnp.int32(0))

        def grp_b(g, carry):
            goff = pl.multiple_of(g * L, 8)
            rem = n_upd - g * L
            rvec = loc_v[pl.ds(goff, L)]
            svec = src_v[pl.ds(goff, L)]
            valid = iota < rem
            svec = jnp.where(valid, svec, 0)
            g_x = pltpu.make_async_copy(x_h.at[svec], gxb, sem_g)
            g_l = pltpu.make_async_copy(lnp_h.at[svec], glb, sem_g)
            g_x.start(); g_l.start()
            vals = plsc.load_gather(byn_v, [svec])
            rsafe = jnp.where(valid, rvec, CH)
            plsc.store_scatter(cby[b], [rsafe], vals, mask=valid)
            plsc.store_scatter(cbt[b], [rsafe], tvec, mask=valid)
            g_x.wait(); g_l.wait()
            for jl in range(L):
                @pl.when(jl < rem)
                def _():
                    lr = rvec[jl]
                    for cc in range(D // L):
                        cs = pl.ds(cc * L, L)
                        cpx[b][lr, cs] = gxb[jl, cs]
                    for cc in range(C // L):
                        cs = pl.ds(cc * L, L)
                        cpl[b][lr, cs] = glb[jl, cs]
                    lrv = iota * 0 + lr
                    tail = glb[jl, pl.ds((C // L) * L, L)]
                    plsc.store_scatter(cpl[b], [lrv, (C // L) * L + iota],
                                       tail, mask=iota < (C % L))
            return carry
        lax.fori_loop(0, (n_upd + L - 1) // L, grp_b, 0)

    start_all(in_copies(0, 0))
    start_all(in_copies(1, 1))

    def outer_b(kk, carry):
        for bb in range(NBUF):
            k = kk * NBUF + bb
            wait_all(in_copies(k, bb))
            update(k, bb)
            start_all(out_copies(k, bb))
            bn = (bb + 2) % NBUF

            @pl.when(k >= 1)
            def _():
                wait_all(out_copies(k - 1, bn))

            @pl.when(k + 2 < NTOT)
            def _():
                start_all(in_copies(k + 2, bn))
        return carry
    lax.fori_loop(0, NTOT // NBUF, outer_b, 0)
    wait_all(out_copies(NTOT - 1, (NTOT - 1) % NBUF))


@jax.jit
def _sc_scatter(bx, logits_buf, by_buf, bt_buf, x, lnp, by_new, idx, tvec):
    f = pl.kernel(
        _body,
        out_type=(
            jax.ShapeDtypeStruct((M, D), jnp.float32),
            jax.ShapeDtypeStruct((M,), jnp.int32),
            jax.ShapeDtypeStruct((M,), jnp.int32),
            jax.ShapeDtypeStruct((M, C), jnp.float32),
        ),
        mesh=plsc.VectorSubcoreMesh(core_axis_name="c", subcore_axis_name="s"),
        compiler_params=pltpu.CompilerParams(needs_layout_passes=False),
        scratch_types=[
            pltpu.VMEM((B,), jnp.int32),
            pltpu.VMEM((B,), jnp.int32),
            pltpu.VMEM((RW,), jnp.int32),
            pltpu.VMEM((CAPC,), jnp.int32),
            pltpu.VMEM((CAPC,), jnp.int32),
            pltpu.VMEM((CH, D), jnp.float32),
            pltpu.VMEM((CH, D), jnp.float32),
            pltpu.VMEM((CH, D), jnp.float32),
            pltpu.VMEM((CH, C), jnp.float32),
            pltpu.VMEM((CH, C), jnp.float32),
            pltpu.VMEM((CH, C), jnp.float32),
            pltpu.VMEM((CH + 1,), jnp.int32),
            pltpu.VMEM((CH + 1,), jnp.int32),
            pltpu.VMEM((CH + 1,), jnp.int32),
            pltpu.VMEM((CH + 1,), jnp.int32),
            pltpu.VMEM((CH + 1,), jnp.int32),
            pltpu.VMEM((CH + 1,), jnp.int32),
            pltpu.VMEM((L, D), jnp.float32),
            pltpu.VMEM((L, D), jnp.float32),
            pltpu.VMEM((L,), jnp.int32),
            pltpu.SemaphoreType.DMA,
            pltpu.SemaphoreType.DMA,
            pltpu.SemaphoreType.DMA,
            pltpu.SemaphoreType.DMA,
            pltpu.SemaphoreType.DMA,
            pltpu.SemaphoreType.DMA,
            pltpu.SemaphoreType.DMA,
        ],
    )
    return f(bx, logits_buf, by_buf, bt_buf, x, lnp, by_new, idx, tvec)


def kernel(bx, logits_buf, by_buf, bt_buf, x, logits_new, by_new, idx, t):
    tvec = jnp.full((L,), t, dtype=jnp.int32)
    lnp = jnp.pad(logits_new, ((0, 0), (0, D - C)))
    return _sc_scatter(bx, logits_buf, by_buf, bt_buf, x, lnp,
                       by_new.astype(jnp.int32), idx.astype(jnp.int32), tvec)


# speedup vs baseline: 1.6256x; 1.6256x over previous
import jax
import jax.numpy as jnp
from jax import lax
from jax.experimental import pallas as pl
from jax.experimental.pallas import tpu as pltpu
from jax.experimental.pallas import tpu_sc as plsc

M = 100000
D = 128
B = 16384
C = 100
L = 16
NW = 32
RW = 3200
CH = 160
NBUF = 2
NTOT = 20
NCHUNK = B // L
LVREG = CH // L
CAPC = CH + L


def _body(bx_h, lg_h, by_h, bt_h, x_h, lnp_h, byn_h, idx_h, tv_h,
          out_bx, out_by, out_bt, out_lg,
          idx_v, byn_v, win_v, loc_v, src_v,
          cpx0, cpx1, cpl0, cpl1,
          cby0, cby1, cbt0, cbt1,
          gxb, glb, tbuf,
          si0, si1, so0, so1, sem_g):
    cid = lax.axis_index("c")
    sid = lax.axis_index("s")
    wid = cid * 16 + sid
    lo = wid * RW
    hi = jnp.minimum(lo + RW, M)
    iota = lax.iota(jnp.int32, L)

    cpx = (cpx0, cpx1)
    cpl = (cpl0, cpl1)
    cby = (cby0, cby1)
    cbt = (cbt0, cbt1)
    sin = (si0, si1)
    sout = (so0, so1)

    pltpu.sync_copy(idx_h, idx_v)
    pltpu.sync_copy(byn_h, byn_v)
    pltpu.sync_copy(tv_h, tbuf)
    tvec = tbuf[...]

    neg1 = jnp.full((L,), -1, jnp.int32)

    def init_b(k, carry):
        win_v[pl.ds(k * L, L)] = neg1
        return carry
    lax.fori_loop(0, RW // L, init_b, 0)

    def scan_b(k, carry):
        v = idx_v[pl.ds(k * L, L)]
        m = (v >= lo) & (v < hi)
        cnt = jnp.sum(m.astype(jnp.int32))
        local = jnp.where(m, v - lo, 0)
        gi = k * L + iota

        @pl.when(cnt == 1)
        def _():
            plsc.store_scatter(win_v, [local], gi, mask=m)

        @pl.when(cnt > 1)
        def _():
            for j in range(L):
                plsc.store_scatter(win_v, [local], gi, mask=m & (iota == j))
        return carry
    lax.fori_loop(0, NCHUNK, scan_b, 0)

    def chunk_base(k):
        return pl.multiple_of(jnp.minimum(lo + k * CH, hi - CH), 8)

    def in_copies(k, b):
        r = chunk_base(k)
        return (
            pltpu.make_async_copy(bx_h.at[pl.ds(r, CH)], cpx[b], sin[b]),
            pltpu.make_async_copy(lg_h.at[pl.ds(r, CH)], cpl[b], sin[b]),
            pltpu.make_async_copy(by_h.at[pl.ds(r, CH)],
                                  cby[b].at[pl.ds(0, CH)], sin[b]),
            pltpu.make_async_copy(bt_h.at[pl.ds(r, CH)],
                                  cbt[b].at[pl.ds(0, CH)], sin[b]),
        )

    def out_copies(k, b):
        r = chunk_base(k)
        return (
            pltpu.make_async_copy(cpx[b], out_bx.at[pl.ds(r, CH)], sout[b]),
            pltpu.make_async_copy(cpl[b], out_lg.at[pl.ds(r, CH)], sout[b]),
            pltpu.make_async_copy(cby[b].at[pl.ds(0, CH)],
                                  out_by.at[pl.ds(r, CH)], sout[b]),
            pltpu.make_async_copy(cbt[b].at[pl.ds(0, CH)],
                                  out_bt.at[pl.ds(r, CH)], sout[b]),
        )

    def start_all(cps):
        for cp in cps:
            cp.start()

    def wait_all(cps):
        for cp in cps:
            cp.wait()

    def update(k, b):
        lb = chunk_base(k) - lo

        def comp_b(j, off):
            w = win_v[pl.ds(lb + j * L, L)]
            mv = w >= 0
            plsc.store_compressed(loc_v.at[pl.ds(off, L)], j * L + iota,
                                  mask=mv)
            plsc.store_compressed(src_v.at[pl.ds(off, L)], w, mask=mv)
            return off + jnp.sum(mv.astype(jnp.int32))
        n_upd = lax.fori_loop(0, LVREG, comp_b, jnp.int32(0))

        def grp_b(g, carry):
            goff = pl.multiple_of(g * L, 8)
            rem = n_upd - g * L
            rvec = loc_v[pl.ds(goff, L)]
            svec = src_v[pl.ds(goff, L)]
            valid = iota < rem
            svec = jnp.where(valid, svec, 0)
            g_x = pltpu.make_async_copy(x_h.at[svec], gxb, sem_g)
            g_l = pltpu.make_async_copy(lnp_h.at[svec], glb, sem_g)
            g_x.start(); g_l.start()
            vals = plsc.load_gather(byn_v, [svec])
            rsafe = jnp.where(valid, rvec, CH)
            plsc.store_scatter(cby[b], [rsafe], vals, mask=valid)
            plsc.store_scatter(cbt[b], [rsafe], tvec, mask=valid)
            g_x.wait(); g_l.wait()
            for jl in range(L):
                @pl.when(jl < rem)
                def _():
                    lr = rvec[jl]
                    for cc in range(D // L):
                        cs = pl.ds(cc * L, L)
                        cpx[b][lr, cs] = gxb[jl, cs]
                    for cc in range(C // L):
                        cs = pl.ds(cc * L, L)
                        cpl[b][lr, cs] = glb[jl, cs]
                    lrv = iota * 0 + lr
                    tail = glb[jl, pl.ds((C // L) * L, L)]
                    plsc.store_scatter(cpl[b], [lrv, (C // L) * L + iota],
                                       tail, mask=iota < (C % L))
            return carry
        lax.fori_loop(0, (n_upd + L - 1) // L, grp_b, 0)

    start_all(in_copies(0, 0))

    def outer_b(kk, carry):
        for bb in range(NBUF):
            k = kk * NBUF + bb
            bn = 1 - bb
            wait_all(in_copies(k, bb))
            update(k, bb)
            start_all(out_copies(k, bb))
            @pl.when(k >= 1)
            def _():
                wait_all(out_copies(k - 1, bn))

            @pl.when(k + 1 < NTOT)
            def _():
                start_all(in_copies(k + 1, bn))
        return carry
    lax.fori_loop(0, NTOT // NBUF, outer_b, 0)
    wait_all(out_copies(NTOT - 1, (NTOT - 1) % NBUF))


@jax.jit
def _sc_scatter(bx, logits_buf, by_buf, bt_buf, x, lnp, by_new, idx, tvec):
    f = pl.kernel(
        _body,
        out_type=(
            jax.ShapeDtypeStruct((M, D), jnp.float32),
            jax.ShapeDtypeStruct((M,), jnp.int32),
            jax.ShapeDtypeStruct((M,), jnp.int32),
            jax.ShapeDtypeStruct((M, C), jnp.float32),
        ),
        mesh=plsc.VectorSubcoreMesh(core_axis_name="c", subcore_axis_name="s"),
        compiler_params=pltpu.CompilerParams(needs_layout_passes=False),
        scratch_types=[
            pltpu.VMEM((B,), jnp.int32),
            pltpu.VMEM((B,), jnp.int32),
            pltpu.VMEM((RW,), jnp.int32),
            pltpu.VMEM((CAPC,), jnp.int32),
            pltpu.VMEM((CAPC,), jnp.int32),
            pltpu.VMEM((CH, D), jnp.float32),
            pltpu.VMEM((CH, D), jnp.float32),
            pltpu.VMEM((CH, C), jnp.float32),
            pltpu.VMEM((CH, C), jnp.float32),
            pltpu.VMEM((CH + 1,), jnp.int32),
            pltpu.VMEM((CH + 1,), jnp.int32),
            pltpu.VMEM((CH + 1,), jnp.int32),
            pltpu.VMEM((CH + 1,), jnp.int32),
            pltpu.VMEM((L, D), jnp.float32),
            pltpu.VMEM((L, D), jnp.float32),
            pltpu.VMEM((L,), jnp.int32),
            pltpu.SemaphoreType.DMA,
            pltpu.SemaphoreType.DMA,
            pltpu.SemaphoreType.DMA,
            pltpu.SemaphoreType.DMA,
            pltpu.SemaphoreType.DMA,
        ],
    )
    return f(bx, logits_buf, by_buf, bt_buf, x, lnp, by_new, idx, tvec)


def kernel(bx, logits_buf, by_buf, bt_buf, x, logits_new, by_new, idx, t):
    tvec = jnp.full((L,), t, dtype=jnp.int32)
    lnp = jnp.pad(logits_new, ((0, 0), (0, D - C)))
    return _sc_scatter(bx, logits_buf, by_buf, bt_buf, x, lnp,
                       by_new.astype(jnp.int32), idx.astype(jnp.int32), tvec)


# speedup vs baseline: 2.2642x; 1.3928x over previous
import jax
import jax.numpy as jnp
from jax import lax
from jax.experimental import pallas as pl
from jax.experimental.pallas import tpu as pltpu
from jax.experimental.pallas import tpu_sc as plsc

M = 100000
D = 128
B = 16384
C = 100
L = 16
NW = 32
RW = 3200
CHB = 800
NBY = RW // CHB
NCHUNK = B // L
CAP = RW + L
BM = 2000


def _copy_body(i_ref, o_ref):
    o_ref[...] = i_ref[...]


def _tc_copy(a):
    return pl.pallas_call(
        _copy_body,
        out_shape=jax.ShapeDtypeStruct(a.shape, a.dtype),
        grid=(M // BM,),
        in_specs=[pl.BlockSpec((BM, a.shape[1]), lambda i: (i, 0))],
        out_specs=pl.BlockSpec((BM, a.shape[1]), lambda i: (i, 0)),
    )(a)


def _pad_body(i_ref, o_ref):
    o_ref[:, pl.ds(0, C)] = i_ref[...]


def _tc_pad(a):
    return pl.pallas_call(
        _pad_body,
        out_shape=jax.ShapeDtypeStruct((M, D), jnp.float32),
        grid=(M // BM,),
        in_specs=[pl.BlockSpec((BM, C), lambda i: (i, 0))],
        out_specs=pl.BlockSpec((BM, D), lambda i: (i, 0)),
    )(a)


def _slice_body(i_ref, o_ref):
    o_ref[...] = i_ref[:, pl.ds(0, C)]


def _tc_slice(a):
    return pl.pallas_call(
        _slice_body,
        out_shape=jax.ShapeDtypeStruct((M, C), jnp.float32),
        grid=(M // BM,),
        in_specs=[pl.BlockSpec((BM, D), lambda i: (i, 0))],
        out_specs=pl.BlockSpec((BM, C), lambda i: (i, 0)),
    )(a)


def _body(rbx, rlgp, by_h, bt_h, x_h, lnp_h, byn_h, idx_h, tv_h,
          out_by, out_bt,
          idx_v, byn_v, win_v, rows_v, src_v,
          cby, cbt, gx0, gx1, gl0, gl1, tbuf,
          sg0, sg1, ss0, ss1):
    cid = lax.axis_index("c")
    sid = lax.axis_index("s")
    wid = cid * 16 + sid
    lo = wid * RW
    hi = jnp.minimum(lo + RW, M)
    iota = lax.iota(jnp.int32, L)

    gx = (gx0, gx1)
    gl = (gl0, gl1)
    sg = (sg0, sg1)
    ss = (ss0, ss1)

    pltpu.sync_copy(idx_h, idx_v)
    pltpu.sync_copy(byn_h, byn_v)
    pltpu.sync_copy(tv_h, tbuf)
    tvec = tbuf[...]

    neg1 = jnp.full((L,), -1, jnp.int32)

    def init_b(k, carry):
        win_v[pl.ds(k * L, L)] = neg1
        return carry
    lax.fori_loop(0, RW // L, init_b, 0)

    def scan_b(k, carry):
        v = idx_v[pl.ds(k * L, L)]
        m = (v >= lo) & (v < hi)
        cnt = jnp.sum(m.astype(jnp.int32))
        local = jnp.where(m, v - lo, 0)
        gi = k * L + iota

        @pl.when(cnt == 1)
        def _():
            plsc.store_scatter(win_v, [local], gi, mask=m)

        @pl.when(cnt > 1)
        def _():
            for j in range(L):
                plsc.store_scatter(win_v, [local], gi, mask=m & (iota == j))
        return carry
    lax.fori_loop(0, NCHUNK, scan_b, 0)

    def byt_b(k, carry):
        r = pl.multiple_of(jnp.minimum(lo + k * CHB, hi - CHB), 8)
        lb = r - lo
        pltpu.sync_copy(by_h.at[pl.ds(r, CHB)], cby.at[pl.ds(0, CHB)])
        pltpu.sync_copy(bt_h.at[pl.ds(r, CHB)], cbt.at[pl.ds(0, CHB)])

        def upd_b(j, carry2):
            w = win_v[pl.ds(lb + j * L, L)]
            mv = w >= 0
            vals = plsc.load_gather(byn_v, [jnp.where(mv, w, 0)])
            rsafe = jnp.where(mv, j * L + iota, CHB)
            plsc.store_scatter(cby, [rsafe], vals, mask=mv)
            plsc.store_scatter(cbt, [rsafe], tvec, mask=mv)
            return carry2
        lax.fori_loop(0, CHB // L, upd_b, 0)
        pltpu.sync_copy(cby.at[pl.ds(0, CHB)], out_by.at[pl.ds(r, CHB)])
        pltpu.sync_copy(cbt.at[pl.ds(0, CHB)], out_bt.at[pl.ds(r, CHB)])
        return carry
    lax.fori_loop(0, NBY, byt_b, 0)

    def comp_b(j, off):
        w = win_v[pl.ds(j * L, L)]
        mv = w >= 0
        plsc.store_compressed(rows_v.at[pl.ds(off, L)], lo + j * L + iota,
                              mask=mv)
        plsc.store_compressed(src_v.at[pl.ds(off, L)], w, mask=mv)
        return off + jnp.sum(mv.astype(jnp.int32))
    n_upd = lax.fori_loop(0, RW // L, comp_b, jnp.int32(0))
    n_g = (n_upd + L - 1) // L

    def drain(bb):
        pltpu.make_async_copy(gx[bb], rbx.at[iota], ss[bb]).wait()
        pltpu.make_async_copy(gl[bb], rlgp.at[iota], ss[bb]).wait()

    def pair_b(p, carry):
        for bb in range(2):
            g = p * 2 + bb

            @pl.when(g < n_g)
            def _():
                goff = pl.multiple_of(g * L, 8)
                rem = n_upd - g * L
                rvec = rows_v[pl.ds(goff, L)]
                svec = src_v[pl.ds(goff, L)]
                valid = iota < rem
                r0 = jnp.min(jnp.where(valid, rvec, 2147483647))
                s0 = jnp.min(jnp.where(valid, svec, 2147483647))
                rvec = jnp.where(valid, rvec, r0)
                svec = jnp.where(valid, svec, s0)

                @pl.when(g >= 2)
                def _():
                    drain(bb)
                c_x = pltpu.make_async_copy(x_h.at[svec], gx[bb], sg[bb])
                c_l = pltpu.make_async_copy(lnp_h.at[svec], gl[bb], sg[bb])
                c_x.start(); c_l.start()
                c_x.wait(); c_l.wait()
                pltpu.make_async_copy(gx[bb], rbx.at[rvec], ss[bb]).start()
                pltpu.make_async_copy(gl[bb], rlgp.at[rvec], ss[bb]).start()
        return carry
    lax.fori_loop(0, (n_g + 1) // 2, pair_b, 0)

    @pl.when(n_g >= 1)
    def _():
        drain(0)

    @pl.when(n_g >= 2)
    def _():
        drain(1)


@jax.jit
def _run(bx, logits_buf, by_buf, bt_buf, x, lnp, by_new, idx, tvec):
    cbx = _tc_copy(bx)
    clgp = _tc_pad(logits_buf)
    rbx = jax.new_ref(cbx)
    rlgp = jax.new_ref(clgp)
    f = pl.kernel(
        _body,
        out_type=(
            jax.ShapeDtypeStruct((M,), jnp.int32),
            jax.ShapeDtypeStruct((M,), jnp.int32),
        ),
        mesh=plsc.VectorSubcoreMesh(core_axis_name="c", subcore_axis_name="s"),
        compiler_params=pltpu.CompilerParams(needs_layout_passes=False),
        scratch_types=[
            pltpu.VMEM((B,), jnp.int32),
            pltpu.VMEM((B,), jnp.int32),
            pltpu.VMEM((RW,), jnp.int32),
            pltpu.VMEM((CAP,), jnp.int32),
            pltpu.VMEM((CAP,), jnp.int32),
            pltpu.VMEM((CHB + 1,), jnp.int32),
            pltpu.VMEM((CHB + 1,), jnp.int32),
            pltpu.VMEM((L, D), jnp.float32),
            pltpu.VMEM((L, D), jnp.float32),
            pltpu.VMEM((L, D), jnp.float32),
            pltpu.VMEM((L, D), jnp.float32),
            pltpu.VMEM((L,), jnp.int32),
            pltpu.SemaphoreType.DMA,
            pltpu.SemaphoreType.DMA,
            pltpu.SemaphoreType.DMA,
            pltpu.SemaphoreType.DMA,
        ],
    )
    out_by, out_bt = f(rbx, rlgp, by_buf, bt_buf, x, lnp, by_new, idx, tvec)
    out_bx = rbx[...]
    out_lg = _tc_slice(rlgp[...])
    return out_bx, out_by, out_bt, out_lg


def kernel(bx, logits_buf, by_buf, bt_buf, x, logits_new, by_new, idx, t):
    tvec = jnp.full((L,), t, dtype=jnp.int32)
    lnp = jnp.pad(logits_new, ((0, 0), (0, D - C)))
    return _run(bx, logits_buf, by_buf, bt_buf, x, lnp,
                by_new.astype(jnp.int32), idx.astype(jnp.int32), tvec)


# speedup vs baseline: 2.2718x; 1.0034x over previous
import jax
import jax.numpy as jnp
from jax import lax
from jax.experimental import pallas as pl
from jax.experimental.pallas import tpu as pltpu
from jax.experimental.pallas import tpu_sc as plsc

M = 100000
D = 128
B = 16384
C = 100
L = 16
NW = 32
RW = 3200
CHB = 800
NBY = RW // CHB
NCHUNK = B // L
CAP = RW + L
BM = 2000


def _copy_body(i_ref, o_ref):
    o_ref[...] = i_ref[...]


def _tc_copy(a):
    return pl.pallas_call(
        _copy_body,
        out_shape=jax.ShapeDtypeStruct(a.shape, a.dtype),
        grid=(M // BM,),
        in_specs=[pl.BlockSpec((BM, a.shape[1]), lambda i: (i, 0))],
        out_specs=pl.BlockSpec((BM, a.shape[1]), lambda i: (i, 0)),
    )(a)


def _pad_body(i_ref, o_ref):
    o_ref[:, pl.ds(0, C)] = i_ref[...]


def _tc_pad(a):
    return pl.pallas_call(
        _pad_body,
        out_shape=jax.ShapeDtypeStruct((M, D), jnp.float32),
        grid=(M // BM,),
        in_specs=[pl.BlockSpec((BM, C), lambda i: (i, 0))],
        out_specs=pl.BlockSpec((BM, D), lambda i: (i, 0)),
    )(a)


def _slice_body(i_ref, o_ref):
    o_ref[...] = i_ref[:, pl.ds(0, C)]


def _tc_slice(a):
    return pl.pallas_call(
        _slice_body,
        out_shape=jax.ShapeDtypeStruct((M, C), jnp.float32),
        grid=(M // BM,),
        in_specs=[pl.BlockSpec((BM, D), lambda i: (i, 0))],
        out_specs=pl.BlockSpec((BM, C), lambda i: (i, 0)),
    )(a)


def _body(rbx, rlgp, by_h, bt_h, x_h, lnp_h, byn_h, idx_h, tv_h,
          out_by, out_bt,
          idx_v, byn_v, win_v, rows_v, src_v,
          cby, cbt, gx0, gx1, gl0, gl1, tbuf,
          sg0, sg1, ss0, ss1):
    cid = lax.axis_index("c")
    sid = lax.axis_index("s")
    wid = cid * 16 + sid
    lo = wid * RW
    hi = jnp.minimum(lo + RW, M)
    iota = lax.iota(jnp.int32, L)

    gx = (gx0, gx1)
    gl = (gl0, gl1)
    sg = (sg0, sg1)
    ss = (ss0, ss1)

    pltpu.sync_copy(idx_h, idx_v)
    pltpu.sync_copy(byn_h, byn_v)
    pltpu.sync_copy(tv_h, tbuf)
    tvec = tbuf[...]

    neg1 = jnp.full((L,), -1, jnp.int32)

    def init_b(k, carry):
        win_v[pl.ds(k * L, L)] = neg1
        return carry
    lax.fori_loop(0, RW // L, init_b, 0)

    def scan_b(k, carry):
        v = idx_v[pl.ds(k * L, L)]
        m = (v >= lo) & (v < hi)
        cnt = jnp.sum(m.astype(jnp.int32))
        local = jnp.where(m, v - lo, 0)
        gi = k * L + iota

        @pl.when(cnt == 1)
        def _():
            plsc.store_scatter(win_v, [local], gi, mask=m)

        @pl.when(cnt > 1)
        def _():
            for j in range(L):
                plsc.store_scatter(win_v, [local], gi, mask=m & (iota == j))
        return carry
    lax.fori_loop(0, NCHUNK, scan_b, 0)

    def byt_b(k, carry):
        r = pl.multiple_of(jnp.minimum(lo + k * CHB, hi - CHB), 8)
        lb = r - lo
        pltpu.sync_copy(by_h.at[pl.ds(r, CHB)], cby.at[pl.ds(0, CHB)])
        pltpu.sync_copy(bt_h.at[pl.ds(r, CHB)], cbt.at[pl.ds(0, CHB)])

        def upd_b(j, carry2):
            w = win_v[pl.ds(lb + j * L, L)]
            mv = w >= 0
            vals = plsc.load_gather(byn_v, [jnp.where(mv, w, 0)])
            rsafe = jnp.where(mv, j * L + iota, CHB)
            plsc.store_scatter(cby, [rsafe], vals, mask=mv)
            plsc.store_scatter(cbt, [rsafe], tvec, mask=mv)
            return carry2
        lax.fori_loop(0, CHB // L, upd_b, 0)
        pltpu.sync_copy(cby.at[pl.ds(0, CHB)], out_by.at[pl.ds(r, CHB)])
        pltpu.sync_copy(cbt.at[pl.ds(0, CHB)], out_bt.at[pl.ds(r, CHB)])
        return carry
    lax.fori_loop(0, NBY, byt_b, 0)

    def comp_b(j, off):
        w = win_v[pl.ds(j * L, L)]
        mv = w >= 0
        plsc.store_compressed(rows_v.at[pl.ds(off, L)], lo + j * L + iota,
                              mask=mv)
        plsc.store_compressed(src_v.at[pl.ds(off, L)], w, mask=mv)
        return off + jnp.sum(mv.astype(jnp.int32))
    n_upd = lax.fori_loop(0, RW // L, comp_b, jnp.int32(0))
    n_g = (n_upd + L - 1) // L

    def drain(bb):
        pltpu.make_async_copy(gx[bb], rbx.at[iota], ss[bb]).wait()
        pltpu.make_async_copy(gl[bb], rlgp.at[iota], ss[bb]).wait()

    def pair_b(p, carry):
        for bb in range(2):
            g = p * 2 + bb

            @pl.when(g < n_g)
            def _():
                goff = pl.multiple_of(g * L, 8)
                rem = n_upd - g * L
                rvec = rows_v[pl.ds(goff, L)]
                svec = src_v[pl.ds(goff, L)]
                valid = iota < rem
                rvec = jnp.where(valid, rvec, rvec[0])
                svec = jnp.where(valid, svec, svec[0])

                @pl.when(g >= 2)
                def _():
                    drain(bb)
                c_x = pltpu.make_async_copy(x_h.at[svec], gx[bb], sg[bb])
                c_l = pltpu.make_async_copy(lnp_h.at[svec], gl[bb], sg[bb])
                c_x.start(); c_l.start()
                c_x.wait(); c_l.wait()
                pltpu.make_async_copy(gx[bb], rbx.at[rvec], ss[bb]).start()
                pltpu.make_async_copy(gl[bb], rlgp.at[rvec], ss[bb]).start()
        return carry
    lax.fori_loop(0, (n_g + 1) // 2, pair_b, 0)

    @pl.when(n_g >= 1)
    def _():
        drain(0)

    @pl.when(n_g >= 2)
    def _():
        drain(1)


@jax.jit
def _run(bx, logits_buf, by_buf, bt_buf, x, lnp, by_new, idx, tvec):
    cbx = _tc_copy(bx)
    clgp = _tc_pad(logits_buf)
    rbx = jax.new_ref(cbx)
    rlgp = jax.new_ref(clgp)
    f = pl.kernel(
        _body,
        out_type=(
            jax.ShapeDtypeStruct((M,), jnp.int32),
            jax.ShapeDtypeStruct((M,), jnp.int32),
        ),
        mesh=plsc.VectorSubcoreMesh(core_axis_name="c", subcore_axis_name="s"),
        compiler_params=pltpu.CompilerParams(needs_layout_passes=False),
        scratch_types=[
            pltpu.VMEM((B,), jnp.int32),
            pltpu.VMEM((B,), jnp.int32),
            pltpu.VMEM((RW,), jnp.int32),
            pltpu.VMEM((CAP,), jnp.int32),
            pltpu.VMEM((CAP,), jnp.int32),
            pltpu.VMEM((CHB + 1,), jnp.int32),
            pltpu.VMEM((CHB + 1,), jnp.int32),
            pltpu.VMEM((L, D), jnp.float32),
            pltpu.VMEM((L, D), jnp.float32),
            pltpu.VMEM((L, D), jnp.float32),
            pltpu.VMEM((L, D), jnp.float32),
            pltpu.VMEM((L,), jnp.int32),
            pltpu.SemaphoreType.DMA,
            pltpu.SemaphoreType.DMA,
            pltpu.SemaphoreType.DMA,
            pltpu.SemaphoreType.DMA,
        ],
    )
    out_by, out_bt = f(rbx, rlgp, by_buf, bt_buf, x, lnp, by_new, idx, tvec)
    out_bx = rbx[...]
    out_lg = _tc_slice(rlgp[...])
    return out_bx, out_by, out_bt, out_lg


def kernel(bx, logits_buf, by_buf, bt_buf, x, logits_new, by_new, idx, t):
    tvec = jnp.full((L,), t, dtype=jnp.int32)
    lnp = jnp.pad(logits_new, ((0, 0), (0, D - C)))
    return _run(bx, logits_buf, by_buf, bt_buf, x, lnp,
                by_new.astype(jnp.int32), idx.astype(jnp.int32), tvec)


# speedup vs baseline: 2.6272x; 1.1564x over previous
import jax
import jax.numpy as jnp
from jax import lax
from jax.experimental import pallas as pl
from jax.experimental.pallas import tpu as pltpu
from jax.experimental.pallas import tpu_sc as plsc

M = 100000
D = 128
B = 16384
C = 100
L = 16
NW = 32
RW = 3200
CHB = 800
NBY = RW // CHB
NCHUNK = B // L
CAP = RW + L
BM = 5000


def _copy_body(i_ref, o_ref):
    o_ref[...] = i_ref[...]


def _tc_copy(a):
    return pl.pallas_call(
        _copy_body,
        out_shape=jax.ShapeDtypeStruct(a.shape, a.dtype),
        grid=(M // BM,),
        in_specs=[pl.BlockSpec((BM, a.shape[1]), lambda i: (i, 0))],
        out_specs=pl.BlockSpec((BM, a.shape[1]), lambda i: (i, 0)),
    )(a)


def _pad_body(i_ref, o_ref):
    o_ref[:, pl.ds(0, C)] = i_ref[...]


def _tc_pad(a):
    return pl.pallas_call(
        _pad_body,
        out_shape=jax.ShapeDtypeStruct((M, D), jnp.float32),
        grid=(M // BM,),
        in_specs=[pl.BlockSpec((BM, C), lambda i: (i, 0))],
        out_specs=pl.BlockSpec((BM, D), lambda i: (i, 0)),
    )(a)


def _slice_body(i_ref, o_ref):
    o_ref[...] = i_ref[:, pl.ds(0, C)]


def _tc_slice(a):
    return pl.pallas_call(
        _slice_body,
        out_shape=jax.ShapeDtypeStruct((M, C), jnp.float32),
        grid=(M // BM,),
        in_specs=[pl.BlockSpec((BM, D), lambda i: (i, 0))],
        out_specs=pl.BlockSpec((BM, C), lambda i: (i, 0)),
    )(a)


def _body(rbx, rlgp, by_h, bt_h, x_h, lnp_h, byn_h, idx_h, tv_h,
          out_by, out_bt,
          idx_v, byn_v, win_v, rows_v, src_v,
          cby, cbt, gx0, gx1, gl0, gl1, tbuf,
          sg0, sg1, ss0, ss1):
    cid = lax.axis_index("c")
    sid = lax.axis_index("s")
    wid = cid * 16 + sid
    lo = wid * RW
    hi = jnp.minimum(lo + RW, M)
    iota = lax.iota(jnp.int32, L)

    gx = (gx0, gx1)
    gl = (gl0, gl1)
    sg = (sg0, sg1)
    ss = (ss0, ss1)

    pltpu.sync_copy(idx_h, idx_v)
    pltpu.sync_copy(byn_h, byn_v)
    pltpu.sync_copy(tv_h, tbuf)
    tvec = tbuf[...]

    neg1 = jnp.full((L,), -1, jnp.int32)

    def init_b(k, carry):
        win_v[pl.ds(k * L, L)] = neg1
        return carry
    lax.fori_loop(0, RW // L, init_b, 0)

    def scan_b(k, carry):
        v = idx_v[pl.ds(k * L, L)]
        m = (v >= lo) & (v < hi)
        cnt = jnp.sum(m.astype(jnp.int32))
        local = jnp.where(m, v - lo, 0)
        gi = k * L + iota

        @pl.when(cnt == 1)
        def _():
            plsc.store_scatter(win_v, [local], gi, mask=m)

        @pl.when(cnt > 1)
        def _():
            for j in range(L):
                plsc.store_scatter(win_v, [local], gi, mask=m & (iota == j))
        return carry
    lax.fori_loop(0, NCHUNK, scan_b, 0)

    def byt_b(k, carry):
        r = pl.multiple_of(jnp.minimum(lo + k * CHB, hi - CHB), 8)
        lb = r - lo
        pltpu.sync_copy(by_h.at[pl.ds(r, CHB)], cby.at[pl.ds(0, CHB)])
        pltpu.sync_copy(bt_h.at[pl.ds(r, CHB)], cbt.at[pl.ds(0, CHB)])

        def upd_b(j, carry2):
            w = win_v[pl.ds(lb + j * L, L)]
            mv = w >= 0
            vals = plsc.load_gather(byn_v, [jnp.where(mv, w, 0)])
            rsafe = jnp.where(mv, j * L + iota, CHB)
            plsc.store_scatter(cby, [rsafe], vals, mask=mv)
            plsc.store_scatter(cbt, [rsafe], tvec, mask=mv)
            return carry2
        lax.fori_loop(0, CHB // L, upd_b, 0)
        pltpu.sync_copy(cby.at[pl.ds(0, CHB)], out_by.at[pl.ds(r, CHB)])
        pltpu.sync_copy(cbt.at[pl.ds(0, CHB)], out_bt.at[pl.ds(r, CHB)])
        return carry
    lax.fori_loop(0, NBY, byt_b, 0)

    def comp_b(j, off):
        w = win_v[pl.ds(j * L, L)]
        mv = w >= 0
        plsc.store_compressed(rows_v.at[pl.ds(off, L)], lo + j * L + iota,
                              mask=mv)
        plsc.store_compressed(src_v.at[pl.ds(off, L)], w, mask=mv)
        return off + jnp.sum(mv.astype(jnp.int32))
    n_upd = lax.fori_loop(0, RW // L, comp_b, jnp.int32(0))
    n_g = (n_upd + L - 1) // L

    def drain(bb):
        pltpu.make_async_copy(gx[bb], rbx.at[iota], ss[bb]).wait()
        pltpu.make_async_copy(gl[bb], rlgp.at[iota], ss[bb]).wait()

    def pair_b(p, carry):
        for bb in range(2):
            g = p * 2 + bb

            @pl.when(g < n_g)
            def _():
                goff = pl.multiple_of(g * L, 8)
                rem = n_upd - g * L
                rvec = rows_v[pl.ds(goff, L)]
                svec = src_v[pl.ds(goff, L)]
                valid = iota < rem
                rvec = jnp.where(valid, rvec, rvec[0])
                svec = jnp.where(valid, svec, svec[0])

                @pl.when(g >= 2)
                def _():
                    drain(bb)
                c_x = pltpu.make_async_copy(x_h.at[svec], gx[bb], sg[bb])
                c_l = pltpu.make_async_copy(lnp_h.at[svec], gl[bb], sg[bb])
                c_x.start(); c_l.start()
                c_x.wait(); c_l.wait()
                pltpu.make_async_copy(gx[bb], rbx.at[rvec], ss[bb]).start()
                pltpu.make_async_copy(gl[bb], rlgp.at[rvec], ss[bb]).start()
        return carry
    lax.fori_loop(0, (n_g + 1) // 2, pair_b, 0)

    @pl.when(n_g >= 1)
    def _():
        drain(0)

    @pl.when(n_g >= 2)
    def _():
        drain(1)


@jax.jit
def _run(bx, logits_buf, by_buf, bt_buf, x, lnp, by_new, idx, tvec):
    cbx = _tc_copy(bx)
    clgp = _tc_pad(logits_buf)
    rbx = jax.new_ref(cbx)
    rlgp = jax.new_ref(clgp)
    f = pl.kernel(
        _body,
        out_type=(
            jax.ShapeDtypeStruct((M,), jnp.int32),
            jax.ShapeDtypeStruct((M,), jnp.int32),
        ),
        mesh=plsc.VectorSubcoreMesh(core_axis_name="c", subcore_axis_name="s"),
        compiler_params=pltpu.CompilerParams(needs_layout_passes=False),
        scratch_types=[
            pltpu.VMEM((B,), jnp.int32),
            pltpu.VMEM((B,), jnp.int32),
            pltpu.VMEM((RW,), jnp.int32),
            pltpu.VMEM((CAP,), jnp.int32),
            pltpu.VMEM((CAP,), jnp.int32),
            pltpu.VMEM((CHB + 1,), jnp.int32),
            pltpu.VMEM((CHB + 1,), jnp.int32),
            pltpu.VMEM((L, D), jnp.float32),
            pltpu.VMEM((L, D), jnp.float32),
            pltpu.VMEM((L, D), jnp.float32),
            pltpu.VMEM((L, D), jnp.float32),
            pltpu.VMEM((L,), jnp.int32),
            pltpu.SemaphoreType.DMA,
            pltpu.SemaphoreType.DMA,
            pltpu.SemaphoreType.DMA,
            pltpu.SemaphoreType.DMA,
        ],
    )
    out_by, out_bt = f(rbx, rlgp, by_buf, bt_buf, x, lnp, by_new, idx, tvec)
    out_bx = rbx[...]
    out_lg = _tc_slice(rlgp[...])
    return out_bx, out_by, out_bt, out_lg


def kernel(bx, logits_buf, by_buf, bt_buf, x, logits_new, by_new, idx, t):
    tvec = jnp.full((L,), t, dtype=jnp.int32)
    lnp = jnp.pad(logits_new, ((0, 0), (0, D - C)))
    return _run(bx, logits_buf, by_buf, bt_buf, x, lnp,
                by_new.astype(jnp.int32), idx.astype(jnp.int32), tvec)
